# baseline (speedup 1.0000x reference)
import functools

import jax
import jax.numpy as jnp
from jax import lax
from jax.experimental import pallas as pl
from jax.experimental.pallas import tpu as pltpu
from jax.experimental.pallas import tpu_sc as plsc

_INFO = plsc.get_sparse_core_info()
_NC = _INFO.num_cores
_NS = _INFO.num_subcores
_NW = _NC * _NS
_L = 16


def _body(n_per_w, d, table_hbm, idx_hbm, ord_hbm, out_hbm,
          idx_v, ord_v, srows_v, rows_v, sem):
    wid = lax.axis_index("s") * _NC + lax.axis_index("c")
    base = wid * n_per_w
    pltpu.sync_copy(idx_hbm.at[pl.ds(base, n_per_w)], idx_v)
    pltpu.sync_copy(ord_hbm.at[pl.ds(base, n_per_w)], ord_v)

    iota = lax.iota(jnp.int32, _L)
    ngrp = n_per_w // _L
    for w in range(2):
        goff = w * (ngrp // 2)
        roff = goff * _L

        def grp_body(g, carry, roff=roff):
            ivec = idx_v[pl.ds(roff + g * _L, _L)]
            for lane in range(_L):
                i = ivec[lane]
                r = g * _L + lane
                pltpu.async_copy(
                    table_hbm.at[pl.ds(i, 1)], srows_v.at[pl.ds(r, 1)], sem)
            return carry

        lax.fori_loop(0, ngrp // 2, grp_body, 0)
        pltpu.make_async_copy(
            table_hbm.at[pl.ds(0, n_per_w // 2)], srows_v, sem).wait()

        def pg_body(g, carry, roff=roff):
            pos = ord_v[pl.ds(roff + g * _L, _L)]
            rvec = iota + g * _L

            def col_body(c, carry2):
                cvec = jnp.zeros((_L,), jnp.int32) + c
                x = plsc.load_gather(srows_v, [rvec, cvec])
                plsc.store_scatter(rows_v, [pos, cvec], x)
                return carry2

            lax.fori_loop(0, d, col_body, jnp.int32(0))
            return carry

        lax.fori_loop(0, ngrp // 2, pg_body, jnp.int32(0))

    pltpu.sync_copy(rows_v, out_hbm.at[wid])


def kernel(full_x, indices):
    n = indices.shape[0]
    d = full_x.shape[1]
    n_per_w = n // _NW
    idx2 = indices.astype(jnp.int32).reshape(_NW, n_per_w)
    iota = jax.lax.broadcasted_iota(jnp.int32, idx2.shape, 1)
    sidx, order = jax.lax.sort([idx2, iota], dimension=1, num_keys=1)

    body = functools.partial(_body, n_per_w, d)
    out = pl.kernel(
        body,
        out_type=jax.ShapeDtypeStruct((_NW, n_per_w, d), jnp.float32),
        mesh=plsc.VectorSubcoreMesh(core_axis_name="c", subcore_axis_name="s"),
        scratch_types=[
            pltpu.VMEM((n_per_w,), jnp.int32),
            pltpu.VMEM((n_per_w,), jnp.int32),
            pltpu.VMEM((n_per_w // 2, d), jnp.float32),
            pltpu.VMEM((n_per_w, d), jnp.float32),
            pltpu.SemaphoreType.DMA,
        ],
        compiler_params=pltpu.CompilerParams(
            use_tc_tiling_on_sc=True, needs_layout_passes=False),
    )(full_x, sidx.reshape(-1), order.reshape(-1))
    return out.reshape(n, d)


# speedup vs baseline: 1.1243x; 1.1243x over previous
import functools

import jax
import jax.numpy as jnp
from jax import lax
from jax.experimental import pallas as pl
from jax.experimental.pallas import tpu as pltpu
from jax.experimental.pallas import tpu_sc as plsc

_INFO = plsc.get_sparse_core_info()
_NC = _INFO.num_cores
_NS = _INFO.num_subcores
_NW = _NC * _NS
_L = 16


def _body(n_per_w, d, table_hbm, idx_hbm, out_hbm, idx_v, rows_v, sem):
    wid = lax.axis_index("s") * _NC + lax.axis_index("c")
    base = wid * n_per_w
    pltpu.sync_copy(idx_hbm.at[pl.ds(base, n_per_w)], idx_v)

    def grp_body(g, carry):
        ivec = idx_v[pl.ds(g * _L, _L)]
        for lane in range(_L):
            i = ivec[lane]
            r = g * _L + lane
            pltpu.async_copy(
                table_hbm.at[pl.ds(i, 1)], rows_v.at[pl.ds(r, 1)], sem)
        return carry

    lax.fori_loop(0, n_per_w // _L, grp_body, 0)

    pltpu.make_async_copy(table_hbm.at[pl.ds(0, n_per_w)], rows_v, sem).wait()
    pltpu.sync_copy(rows_v, out_hbm.at[wid])


def kernel(full_x, indices):
    n = indices.shape[0]
    d = full_x.shape[1]
    n_per_w = n // _NW
    idx = indices.astype(jnp.int32)

    body = functools.partial(_body, n_per_w, d)
    out = pl.kernel(
        body,
        out_type=jax.ShapeDtypeStruct((_NW, n_per_w, d), jnp.float32),
        mesh=plsc.VectorSubcoreMesh(core_axis_name="c", subcore_axis_name="s"),
        scratch_types=[
            pltpu.VMEM((n_per_w,), jnp.int32),
            pltpu.VMEM((n_per_w, d), jnp.float32),
            pltpu.SemaphoreType.DMA,
        ],
        compiler_params=pltpu.CompilerParams(
            use_tc_tiling_on_sc=True, needs_layout_passes=False),
    )(full_x, idx)
    return out.reshape(n, d)


# speedup vs baseline: 1.1245x; 1.0002x over previous
import functools

import jax
import jax.numpy as jnp
from jax import lax
from jax.experimental import pallas as pl
from jax.experimental.pallas import tpu as pltpu
from jax.experimental.pallas import tpu_sc as plsc

_NC = 2
_NS = 16
_NW = _NC * _NS
_L = 16


def _body(n_per_w, d, table_hbm, idx_hbm, out_hbm, idx_v, rows_v, sem):
    wid = lax.axis_index("s") * _NC + lax.axis_index("c")
    base = wid * n_per_w
    pltpu.sync_copy(idx_hbm.at[pl.ds(base, n_per_w)], idx_v)

    def grp_body(g, carry):
        ivec = idx_v[pl.ds(g * _L, _L)]
        for lane in range(_L):
            i = ivec[lane]
            r = g * _L + lane
            pltpu.async_copy(
                table_hbm.at[pl.ds(i, 1)], rows_v.at[pl.ds(r, 1)], sem)
        return carry

    lax.fori_loop(0, n_per_w // _L, grp_body, 0)

    pltpu.make_async_copy(table_hbm.at[pl.ds(0, n_per_w)], rows_v, sem).wait()
    pltpu.sync_copy(rows_v, out_hbm.at[wid])


def kernel(full_x, indices):
    info = plsc.get_sparse_core_info()
    assert (info.num_cores, info.num_subcores) == (_NC, _NS)
    n = indices.shape[0]
    d = full_x.shape[1]
    n_per_w = n // _NW
    idx = indices.astype(jnp.int32)

    body = functools.partial(_body, n_per_w, d)
    out = pl.kernel(
        body,
        out_type=jax.ShapeDtypeStruct((_NW, n_per_w, d), jnp.float32),
        mesh=plsc.VectorSubcoreMesh(core_axis_name="c", subcore_axis_name="s"),
        scratch_types=[
            pltpu.VMEM((n_per_w,), jnp.int32),
            pltpu.VMEM((n_per_w, d), jnp.float32),
            pltpu.SemaphoreType.DMA,
        ],
        compiler_params=pltpu.CompilerParams(
            use_tc_tiling_on_sc=True, needs_layout_passes=False),
    )(full_x, idx)
    return out.reshape(n, d)
